# baseline (speedup 1.0000x reference)
import functools

import jax
import jax.numpy as jnp
import numpy as np
from jax import lax
from jax.experimental import pallas as pl
from jax.experimental.pallas import tpu as pltpu
from jax.experimental.pallas import tpu_sc as plsc

BATCH = 4096
IN_DIM = 1024
OUT_DIM = 2048
L = 16

_GATE_COEFFS = np.array(
    [
        [0, 0, 0, 0],
        [0, 0, 0, 1],
        [0, 1, 0, -1],
        [0, 1, 0, 0],
        [0, 0, 1, -1],
        [0, 0, 1, 0],
        [0, 1, 1, -2],
        [0, 1, 1, -1],
        [1, -1, -1, 1],
        [1, -1, -1, 2],
        [1, 0, -1, 0],
        [1, 0, -1, 1],
        [1, -1, 0, 0],
        [1, -1, 0, 1],
        [1, 0, 0, -1],
        [1, 0, 0, 0],
    ],
    dtype=np.float32,
)


def _coeff_body(wt_ref, ct_ref, out_ref):
    wt = wt_ref[...]
    m = jnp.max(wt, axis=0, keepdims=True)
    e = jnp.exp(wt - m)
    sm = e / jnp.sum(e, axis=0, keepdims=True)
    ct = ct_ref[...]
    out_ref[...] = lax.dot_general(
        ct, sm, (((1,), (0,)), ((), ())),
        precision=lax.Precision.HIGHEST,
        preferred_element_type=jnp.float32,
    )


_coeff_call = pl.pallas_call(
    _coeff_body,
    out_shape=jax.ShapeDtypeStruct((4, OUT_DIM), jnp.float32),
)


_R = 16


def _make_sc_main():
    info = plsc.get_sparse_core_info()
    nw = info.num_cores * info.num_subcores
    rows_per_w = BATCH // nw
    nchunk = OUT_DIM // L
    nblocks = rows_per_w // _R
    mesh = plsc.VectorSubcoreMesh(core_axis_name="c", subcore_axis_name="s")

    @functools.partial(
        pl.kernel,
        mesh=mesh,
        out_type=jax.ShapeDtypeStruct((BATCH, OUT_DIM), jnp.float32),
        compiler_params=pltpu.CompilerParams(needs_layout_passes=False),
        scratch_types=[
            pltpu.VMEM((OUT_DIM,), jnp.int32),
            pltpu.VMEM((OUT_DIM,), jnp.int32),
            pltpu.VMEM((4, OUT_DIM), jnp.float32),
            pltpu.VMEM((_R, IN_DIM), jnp.float32),
            pltpu.VMEM((_R, IN_DIM), jnp.float32),
            pltpu.VMEM((_R, OUT_DIM), jnp.float32),
            pltpu.VMEM((_R, OUT_DIM), jnp.float32),
            pltpu.SemaphoreType.DMA,
            pltpu.SemaphoreType.DMA,
            pltpu.SemaphoreType.DMA,
            pltpu.SemaphoreType.DMA,
        ],
    )
    def sc_main(x_hbm, k_hbm, ca_hbm, cb_hbm, out_hbm, ca_v, cb_v, k_v,
                xb0, xb1, ob0, ob1, isem0, isem1, osem0, osem1):
        wid = lax.axis_index("s") * info.num_cores + lax.axis_index("c")
        pltpu.sync_copy(ca_hbm, ca_v)
        pltpu.sync_copy(cb_hbm, cb_v)
        pltpu.sync_copy(k_hbm, k_v)
        base = wid * rows_per_w
        xbufs, obufs = (xb0, xb1), (ob0, ob1)
        isems, osems = (isem0, isem1), (osem0, osem1)

        def x_slice(blk):
            return x_hbm.at[pl.ds(base + blk * _R, _R), :]

        def o_slice(blk):
            return out_hbm.at[pl.ds(base + blk * _R, _R), :]

        row_ids = [jnp.full((L,), r, jnp.int32) for r in range(_R)]

        def compute(xb, ob):
            @plsc.parallel_loop(0, nchunk, unroll=4)
            def _chunk(c):
                off = c * L
                idxa = ca_v[pl.ds(off, L)]
                idxb = cb_v[pl.ds(off, L)]
                k0 = k_v[0, pl.ds(off, L)]
                k1 = k_v[1, pl.ds(off, L)]
                k2 = k_v[2, pl.ds(off, L)]
                k3 = k_v[3, pl.ds(off, L)]
                for r in range(_R):
                    a = plsc.load_gather(xb, [row_ids[r], idxa])
                    b = plsc.load_gather(xb, [row_ids[r], idxb])
                    ob[r, pl.ds(off, L)] = (
                        k0 + a * k1 + b * k2 + (a * b) * k3)

        pltpu.async_copy(x_slice(0), xb0, isem0)
        pltpu.async_copy(x_slice(1), xb1, isem1)

        def iter_body(i, carry):
            for b in range(2):
                blk = 2 * i + b
                pltpu.make_async_copy(x_slice(blk), xbufs[b], isems[b]).wait()

                @pl.when(i > 0)
                def _():
                    pltpu.make_async_copy(
                        obufs[b], o_slice(blk), osems[b]).wait()

                compute(xbufs[b], obufs[b])
                pltpu.async_copy(obufs[b], o_slice(blk), osems[b])

                @pl.when(blk + 2 < nblocks)
                def _():
                    pltpu.async_copy(x_slice(blk + 2), xbufs[b], isems[b])
            return carry

        lax.fori_loop(0, nblocks // 2, iter_body, 0)
        pltpu.make_async_copy(ob0, o_slice(nblocks - 2), osem0).wait()
        pltpu.make_async_copy(ob1, o_slice(nblocks - 1), osem1).wait()

    return sc_main


_sc_main = _make_sc_main()


def kernel(x, weights, conn_a, conn_b):
    coeffs = _coeff_call(weights.T, jnp.asarray(_GATE_COEFFS.T))
    return _sc_main(x, coeffs, conn_a, conn_b)


# speedup vs baseline: 1.8033x; 1.8033x over previous
import functools

import jax
import jax.numpy as jnp
import numpy as np
from jax import lax
from jax.experimental import pallas as pl
from jax.experimental.pallas import tpu as pltpu
from jax.experimental.pallas import tpu_sc as plsc

BATCH = 4096
IN_DIM = 1024
OUT_DIM = 2048
L = 16

_GATE_COEFFS = np.array(
    [
        [0, 0, 0, 0],
        [0, 0, 0, 1],
        [0, 1, 0, -1],
        [0, 1, 0, 0],
        [0, 0, 1, -1],
        [0, 0, 1, 0],
        [0, 1, 1, -2],
        [0, 1, 1, -1],
        [1, -1, -1, 1],
        [1, -1, -1, 2],
        [1, 0, -1, 0],
        [1, 0, -1, 1],
        [1, -1, 0, 0],
        [1, -1, 0, 1],
        [1, 0, 0, -1],
        [1, 0, 0, 0],
    ],
    dtype=np.float32,
)


def _coeff_body(wt_ref, ct_ref, out_ref):
    wt = wt_ref[...]
    m = jnp.max(wt, axis=0, keepdims=True)
    e = jnp.exp(wt - m)
    sm = e / jnp.sum(e, axis=0, keepdims=True)
    ct = ct_ref[...]
    out_ref[...] = lax.dot_general(
        ct, sm, (((1,), (0,)), ((), ())),
        precision=lax.Precision.HIGHEST,
        preferred_element_type=jnp.float32,
    )


_coeff_call = pl.pallas_call(
    _coeff_body,
    out_shape=jax.ShapeDtypeStruct((4, OUT_DIM), jnp.float32),
)


_R = 16


def _make_sc_main():
    info = plsc.get_sparse_core_info()
    nw = info.num_cores * info.num_subcores
    rows_per_w = BATCH // nw
    nchunk = OUT_DIM // L
    nblocks = rows_per_w // _R
    mesh = plsc.VectorSubcoreMesh(core_axis_name="c", subcore_axis_name="s")

    @functools.partial(
        pl.kernel,
        mesh=mesh,
        out_type=jax.ShapeDtypeStruct((BATCH, OUT_DIM), jnp.float32),
        compiler_params=pltpu.CompilerParams(needs_layout_passes=False),
        scratch_types=[
            pltpu.VMEM((OUT_DIM,), jnp.int32),
            pltpu.VMEM((OUT_DIM,), jnp.int32),
            pltpu.VMEM((4, OUT_DIM), jnp.float32),
            pltpu.VMEM((_R, IN_DIM), jnp.float32),
            pltpu.VMEM((_R, IN_DIM), jnp.float32),
            pltpu.VMEM((_R, OUT_DIM), jnp.float32),
            pltpu.VMEM((_R, OUT_DIM), jnp.float32),
            pltpu.SemaphoreType.DMA,
            pltpu.SemaphoreType.DMA,
            pltpu.SemaphoreType.DMA,
            pltpu.SemaphoreType.DMA,
        ],
    )
    def sc_main(x_hbm, k_hbm, ca_hbm, cb_hbm, out_hbm, ca_v, cb_v, k_v,
                xb0, xb1, ob0, ob1, isem0, isem1, osem0, osem1):
        wid = lax.axis_index("s") * info.num_cores + lax.axis_index("c")
        pltpu.sync_copy(ca_hbm, ca_v)
        pltpu.sync_copy(cb_hbm, cb_v)
        pltpu.sync_copy(k_hbm, k_v)
        base = wid * rows_per_w
        xbufs, obufs = (xb0, xb1), (ob0, ob1)
        isems, osems = (isem0, isem1), (osem0, osem1)

        def x_slice(blk):
            return x_hbm.at[pl.ds(base + blk * _R, _R), :]

        def o_slice(blk):
            return out_hbm.at[pl.ds(base + blk * _R, _R), :]

        row_ids = [jnp.full((L,), r, jnp.int32) for r in range(_R)]

        def compute(xb, ob):
            @plsc.parallel_loop(0, nchunk, unroll=1)
            def _chunk(c):
                off = c * L
                idxa = ca_v[pl.ds(off, L)]
                idxb = cb_v[pl.ds(off, L)]
                k0 = k_v[0, pl.ds(off, L)]
                k1 = k_v[1, pl.ds(off, L)]
                k2 = k_v[2, pl.ds(off, L)]
                k3 = k_v[3, pl.ds(off, L)]
                for r in range(_R):
                    a = plsc.load_gather(xb, [row_ids[r], idxa])
                    b = plsc.load_gather(xb, [row_ids[r], idxb])
                    ob[r, pl.ds(off, L)] = (
                        k0 + a * k1 + b * k2 + (a * b) * k3)

        pltpu.async_copy(x_slice(0), xb0, isem0)
        pltpu.async_copy(x_slice(1), xb1, isem1)

        def iter_body(i, carry):
            for b in range(2):
                blk = 2 * i + b
                pltpu.make_async_copy(x_slice(blk), xbufs[b], isems[b]).wait()

                @pl.when(i > 0)
                def _():
                    pltpu.make_async_copy(
                        obufs[b], o_slice(blk), osems[b]).wait()

                compute(xbufs[b], obufs[b])
                pltpu.async_copy(obufs[b], o_slice(blk), osems[b])

                @pl.when(blk + 2 < nblocks)
                def _():
                    pltpu.async_copy(x_slice(blk + 2), xbufs[b], isems[b])
            return carry

        lax.fori_loop(0, nblocks // 2, iter_body, 0)
        pltpu.make_async_copy(ob0, o_slice(nblocks - 2), osem0).wait()
        pltpu.make_async_copy(ob1, o_slice(nblocks - 1), osem1).wait()

    return sc_main


_sc_main = _make_sc_main()


def kernel(x, weights, conn_a, conn_b):
    coeffs = _coeff_call(weights.T, jnp.asarray(_GATE_COEFFS.T))
    return _sc_main(x, coeffs, conn_a, conn_b)


# speedup vs baseline: 1.8181x; 1.0082x over previous
import functools

import jax
import jax.numpy as jnp
import numpy as np
from jax import lax
from jax.experimental import pallas as pl
from jax.experimental.pallas import tpu as pltpu
from jax.experimental.pallas import tpu_sc as plsc

BATCH = 4096
IN_DIM = 1024
OUT_DIM = 2048
L = 16

_GATE_COEFFS = np.array(
    [
        [0, 0, 0, 0],
        [0, 0, 0, 1],
        [0, 1, 0, -1],
        [0, 1, 0, 0],
        [0, 0, 1, -1],
        [0, 0, 1, 0],
        [0, 1, 1, -2],
        [0, 1, 1, -1],
        [1, -1, -1, 1],
        [1, -1, -1, 2],
        [1, 0, -1, 0],
        [1, 0, -1, 1],
        [1, -1, 0, 0],
        [1, -1, 0, 1],
        [1, 0, 0, -1],
        [1, 0, 0, 0],
    ],
    dtype=np.float32,
)


def _coeff_body(wt_ref, ct_ref, ca_ref, cb_ref, out_ref, cab_ref):
    wt = wt_ref[...]
    m = jnp.max(wt, axis=0, keepdims=True)
    e = jnp.exp(wt - m)
    sm = e / jnp.sum(e, axis=0, keepdims=True)
    ct = ct_ref[...]
    k = lax.dot_general(
        ct, sm, (((1,), (0,)), ((), ())),
        precision=lax.Precision.HIGHEST,
        preferred_element_type=jnp.float32,
    )
    kb = lax.bitcast_convert_type(
        k.astype(jnp.bfloat16), jnp.uint16).astype(jnp.uint32)
    packed = jnp.stack([kb[0] | (kb[1] << 16), kb[2] | (kb[3] << 16)])
    out_ref[...] = lax.bitcast_convert_type(packed, jnp.int32)
    cab_ref[...] = ca_ref[...] | (cb_ref[...] << 16)


_coeff_call = pl.pallas_call(
    _coeff_body,
    out_shape=(
        jax.ShapeDtypeStruct((2, OUT_DIM), jnp.int32),
        jax.ShapeDtypeStruct((OUT_DIM,), jnp.int32),
    ),
)


_R = 16


def _make_sc_main():
    info = plsc.get_sparse_core_info()
    nw = info.num_cores * info.num_subcores
    rows_per_w = BATCH // nw
    nchunk = OUT_DIM // L
    nblocks = rows_per_w // _R
    mesh = plsc.VectorSubcoreMesh(core_axis_name="c", subcore_axis_name="s")

    @functools.partial(
        pl.kernel,
        mesh=mesh,
        out_type=jax.ShapeDtypeStruct((BATCH, OUT_DIM), jnp.float32),
        compiler_params=pltpu.CompilerParams(needs_layout_passes=False),
        scratch_types=[
            pltpu.VMEM((OUT_DIM,), jnp.int32),
            pltpu.VMEM((2, OUT_DIM), jnp.int32),
            pltpu.VMEM((_R, IN_DIM), jnp.float32),
            pltpu.VMEM((_R, IN_DIM), jnp.float32),
            pltpu.VMEM((_R, OUT_DIM), jnp.float32),
            pltpu.VMEM((_R, OUT_DIM), jnp.float32),
            pltpu.SemaphoreType.DMA,
            pltpu.SemaphoreType.DMA,
            pltpu.SemaphoreType.DMA,
            pltpu.SemaphoreType.DMA,
        ],
    )
    def sc_main(x_hbm, k_hbm, cab_hbm, out_hbm, cab_v, k_v,
                xb0, xb1, ob0, ob1, isem0, isem1, osem0, osem1):
        wid = lax.axis_index("s") * info.num_cores + lax.axis_index("c")
        pltpu.sync_copy(cab_hbm, cab_v)
        pltpu.sync_copy(k_hbm, k_v)
        base = wid * rows_per_w
        xbufs, obufs = (xb0, xb1), (ob0, ob1)
        isems, osems = (isem0, isem1), (osem0, osem1)

        def x_slice(blk):
            return x_hbm.at[pl.ds(base + blk * _R, _R), :]

        def o_slice(blk):
            return out_hbm.at[pl.ds(base + blk * _R, _R), :]

        row_ids = [jnp.full((L,), r, jnp.int32) for r in range(_R)]

        def compute(xb, ob):
            @plsc.parallel_loop(0, nchunk, unroll=1)
            def _chunk(c):
                off = c * L
                cab = cab_v[pl.ds(off, L)]
                idxa = cab & 0xFFFF
                idxb = lax.shift_right_logical(cab, 16)
                w01 = k_v[0, pl.ds(off, L)]
                w23 = k_v[1, pl.ds(off, L)]
                k0 = plsc.bitcast(lax.shift_left(w01, 16), jnp.float32)
                k1 = plsc.bitcast(w01 & jnp.int32(-65536), jnp.float32)
                k2 = plsc.bitcast(lax.shift_left(w23, 16), jnp.float32)
                k3 = plsc.bitcast(w23 & jnp.int32(-65536), jnp.float32)
                for r in range(_R):
                    a = plsc.load_gather(xb, [row_ids[r], idxa])
                    b = plsc.load_gather(xb, [row_ids[r], idxb])
                    ob[r, pl.ds(off, L)] = (
                        k0 + a * k1 + b * k2 + (a * b) * k3)

        pltpu.async_copy(x_slice(0), xb0, isem0)
        pltpu.async_copy(x_slice(1), xb1, isem1)

        def iter_body(i, carry):
            for b in range(2):
                blk = 2 * i + b
                pltpu.make_async_copy(x_slice(blk), xbufs[b], isems[b]).wait()

                @pl.when(i > 0)
                def _():
                    pltpu.make_async_copy(
                        obufs[b], o_slice(blk), osems[b]).wait()

                compute(xbufs[b], obufs[b])
                pltpu.async_copy(obufs[b], o_slice(blk), osems[b])

                @pl.when(blk + 2 < nblocks)
                def _():
                    pltpu.async_copy(x_slice(blk + 2), xbufs[b], isems[b])
            return carry

        lax.fori_loop(0, nblocks // 2, iter_body, 0)
        pltpu.make_async_copy(ob0, o_slice(nblocks - 2), osem0).wait()
        pltpu.make_async_copy(ob1, o_slice(nblocks - 1), osem1).wait()

    return sc_main


_sc_main = _make_sc_main()


def kernel(x, weights, conn_a, conn_b):
    coeffs, cab = _coeff_call(
        weights.T, jnp.asarray(_GATE_COEFFS.T), conn_a, conn_b)
    return _sc_main(x, coeffs, cab)


# speedup vs baseline: 1.8553x; 1.0204x over previous
import functools

import jax
import jax.numpy as jnp
import numpy as np
from jax import lax
from jax.experimental import pallas as pl
from jax.experimental.pallas import tpu as pltpu
from jax.experimental.pallas import tpu_sc as plsc

BATCH = 4096
IN_DIM = 1024
OUT_DIM = 2048
L = 16

_GATE_COEFFS = np.array(
    [
        [0, 0, 0, 0],
        [0, 0, 0, 1],
        [0, 1, 0, -1],
        [0, 1, 0, 0],
        [0, 0, 1, -1],
        [0, 0, 1, 0],
        [0, 1, 1, -2],
        [0, 1, 1, -1],
        [1, -1, -1, 1],
        [1, -1, -1, 2],
        [1, 0, -1, 0],
        [1, 0, -1, 1],
        [1, -1, 0, 0],
        [1, -1, 0, 1],
        [1, 0, 0, -1],
        [1, 0, 0, 0],
    ],
    dtype=np.float32,
)


def _coeff_body(wt_ref, ct_ref, ca_ref, cb_ref, out_ref, cab_ref):
    wt = wt_ref[...]
    m = jnp.max(wt, axis=0, keepdims=True)
    e = jnp.exp(wt - m)
    sm = e / jnp.sum(e, axis=0, keepdims=True)
    ct = ct_ref[...]
    k = lax.dot_general(
        ct, sm, (((1,), (0,)), ((), ())),
        precision=lax.Precision.HIGHEST,
        preferred_element_type=jnp.float32,
    )
    kb = lax.bitcast_convert_type(
        k.astype(jnp.bfloat16), jnp.uint16).astype(jnp.uint32)
    packed = jnp.stack([kb[0] | (kb[1] << 16), kb[2] | (kb[3] << 16)])
    out_ref[...] = lax.bitcast_convert_type(packed, jnp.int32)
    cab_ref[...] = ca_ref[...] | (cb_ref[...] << 16)


_coeff_call = pl.pallas_call(
    _coeff_body,
    out_shape=(
        jax.ShapeDtypeStruct((2, OUT_DIM), jnp.int32),
        jax.ShapeDtypeStruct((OUT_DIM,), jnp.int32),
    ),
)


_R = 16


def _make_sc_main():
    info = plsc.get_sparse_core_info()
    nw = info.num_cores * info.num_subcores
    rows_per_w = BATCH // nw
    nchunk = OUT_DIM // L
    nblocks = rows_per_w // _R
    mesh = plsc.VectorSubcoreMesh(core_axis_name="c", subcore_axis_name="s")

    @functools.partial(
        pl.kernel,
        mesh=mesh,
        out_type=jax.ShapeDtypeStruct((BATCH, OUT_DIM), jnp.float32),
        compiler_params=pltpu.CompilerParams(needs_layout_passes=False),
        scratch_types=[
            pltpu.VMEM((OUT_DIM,), jnp.int32),
            pltpu.VMEM((2, OUT_DIM), jnp.int32),
            pltpu.VMEM((_R, IN_DIM), jnp.float32),
            pltpu.VMEM((_R, IN_DIM), jnp.float32),
            pltpu.VMEM((_R, OUT_DIM), jnp.float32),
            pltpu.VMEM((_R, OUT_DIM), jnp.float32),
            pltpu.SemaphoreType.DMA,
            pltpu.SemaphoreType.DMA,
            pltpu.SemaphoreType.DMA,
            pltpu.SemaphoreType.DMA,
        ],
    )
    def sc_main(x_hbm, k_hbm, cab_hbm, out_hbm, cab_v, k_v,
                xb0, xb1, ob0, ob1, isem0, isem1, osem0, osem1):
        wid = lax.axis_index("s") * info.num_cores + lax.axis_index("c")
        pltpu.sync_copy(cab_hbm, cab_v)
        pltpu.sync_copy(k_hbm, k_v)
        base = wid * rows_per_w
        xbufs, obufs = (xb0, xb1), (ob0, ob1)
        isems, osems = (isem0, isem1), (osem0, osem1)

        def x_slice(blk):
            return x_hbm.at[pl.ds(base + blk * _R, _R), :]

        def o_slice(blk):
            return out_hbm.at[pl.ds(base + blk * _R, _R), :]

        row_ids = [jnp.full((L,), r, jnp.int32) for r in range(_R)]

        def compute(xb, ob):
            @plsc.parallel_loop(0, nchunk, unroll=1)
            def _chunk(c):
                off = c * L
                cab = cab_v[pl.ds(off, L)]
                idxa = cab & 0xFFFF
                idxb = lax.shift_right_logical(cab, 16)
                w01 = k_v[0, pl.ds(off, L)]
                w23 = k_v[1, pl.ds(off, L)]
                k0 = plsc.bitcast(lax.shift_left(w01, 16), jnp.float32)
                k1 = plsc.bitcast(w01 & jnp.int32(-65536), jnp.float32)
                k2 = plsc.bitcast(lax.shift_left(w23, 16), jnp.float32)
                k3 = plsc.bitcast(w23 & jnp.int32(-65536), jnp.float32)
                for r in range(_R):
                    a = plsc.load_gather(xb, [row_ids[r], idxa])
                    b = plsc.load_gather(xb, [row_ids[r], idxb])
                    ob[r, pl.ds(off, L)] = k0 + a * k1
                    plsc.addupdate(ob.at[r, pl.ds(off, L)], b * (k2 + a * k3))

        pltpu.async_copy(x_slice(0), xb0, isem0)
        pltpu.async_copy(x_slice(1), xb1, isem1)

        def iter_body(i, carry):
            for b in range(2):
                blk = 2 * i + b
                pltpu.make_async_copy(x_slice(blk), xbufs[b], isems[b]).wait()

                @pl.when(i > 0)
                def _():
                    pltpu.make_async_copy(
                        obufs[b], o_slice(blk), osems[b]).wait()

                compute(xbufs[b], obufs[b])
                pltpu.async_copy(obufs[b], o_slice(blk), osems[b])

                @pl.when(blk + 2 < nblocks)
                def _():
                    pltpu.async_copy(x_slice(blk + 2), xbufs[b], isems[b])
            return carry

        lax.fori_loop(0, nblocks // 2, iter_body, 0)
        pltpu.make_async_copy(ob0, o_slice(nblocks - 2), osem0).wait()
        pltpu.make_async_copy(ob1, o_slice(nblocks - 1), osem1).wait()

    return sc_main


_sc_main = _make_sc_main()


def kernel(x, weights, conn_a, conn_b):
    coeffs, cab = _coeff_call(
        weights.T, jnp.asarray(_GATE_COEFFS.T), conn_a, conn_b)
    return _sc_main(x, coeffs, cab)
